# baseline (speedup 1.0000x reference)
import functools
from itertools import combinations

import jax
import jax.numpy as jnp
from jax import lax
from jax.experimental import pallas as pl
from jax.experimental.pallas import tpu as pltpu
from jax.experimental.pallas import tpu_sc as plsc

_NUM_CLASSES = 7
_MAX_SET_SIZE = 2


def _powerset_members():
    mapping = [()]
    for set_size in range(1, _MAX_SET_SIZE + 1):
        mapping.extend(combinations(range(_NUM_CLASSES), set_size))
    members = [[] for _ in range(_NUM_CLASSES)]
    for k, classes in enumerate(mapping):
        for c in classes:
            members[c].append(k)
    return len(mapping), members


_K, _MEMBERS = _powerset_members()


def _tree_reduce(vals, op):
    vals = list(vals)
    while len(vals) > 1:
        nxt = [op(vals[i], vals[i + 1]) for i in range(0, len(vals) - 1, 2)]
        if len(vals) % 2:
            nxt.append(vals[-1])
        vals = nxt
    return vals[0]


def kernel(powerset, mapping_matrix):
    B, F, K = powerset.shape
    C = mapping_matrix.shape[1]
    assert K == _K and C == _NUM_CLASSES
    N = B * F
    x = powerset.reshape(N * K)

    NC, NS, L = 2, 16, 16
    NW = NC * NS
    RW = N // NW
    nblk = RW // L

    @functools.partial(
        pl.kernel,
        out_type=jax.ShapeDtypeStruct((N * C,), jnp.float32),
        mesh=plsc.VectorSubcoreMesh(core_axis_name="c", subcore_axis_name="s",
                                    num_cores=NC, num_subcores=NS),
        scratch_types=[
            pltpu.VMEM((RW * K,), jnp.float32),
            pltpu.VMEM((RW * C,), jnp.float32),
        ],
        compiler_params=pltpu.CompilerParams(needs_layout_passes=False),
    )
    def _run(x_hbm, out_hbm, in_v, out_v):
        wid = lax.axis_index("s") * NC + lax.axis_index("c")
        pltpu.sync_copy(x_hbm.at[pl.ds(wid * (RW * K), RW * K)], in_v)
        lanes = lax.iota(jnp.int32, L)

        def block(b, carry):
            ridx = lanes * K + b * (L * K)
            v = [plsc.load_gather(in_v, [ridx + k]) for k in range(K)]
            m = _tree_reduce(v, jnp.maximum)
            e = [jnp.exp(vk - m) for vk in v]
            r = 1.0 / _tree_reduce(e, lambda a, b_: a + b_)
            oidx = lanes * C + b * (L * C)
            for c in range(C):
                acc = _tree_reduce([e[k] for k in _MEMBERS[c]],
                                   lambda a, b_: a + b_)
                plsc.store_scatter(out_v, [oidx + c], acc * r)
            return carry

        lax.fori_loop(0, nblk, block, 0)
        pltpu.sync_copy(out_v, out_hbm.at[pl.ds(wid * (RW * C), RW * C)])

    return _run(x).reshape(B, F, C)
